# baseline (speedup 1.0000x reference)
import functools
import math

import jax
import jax.numpy as jnp
from jax import lax
from jax.experimental import pallas as pl
from jax.experimental.pallas import tpu as pltpu
from jax.experimental.pallas import tpu_sc as plsc

_B, _N, _D, _L = 4, 2048, 128, 128
_RANK, _DH, _BOT = 32, 256, 64
_K = 20
_TAU = 1.0
_BM = 256
_NEG = -3.0e38


def _prep_body(X_ref, hist_ref, W1_ref, b1_ref, W2_ref, b2_ref, Ws_ref,
               Wr_ref, Bm_ref, Xm_ref, SB_ref, R_ref, nrm_ref):
    X = X_ref[...]
    h = jnp.maximum(jnp.dot(X, W1_ref[...], preferred_element_type=jnp.float32)
                    + b1_ref[...], 0.0)
    Xm = X + jnp.dot(h, W2_ref[...], preferred_element_type=jnp.float32) + b2_ref[...]
    Xm_ref[...] = Xm
    S = jnp.dot(Xm, Ws_ref[...], preferred_element_type=jnp.float32)
    SB_ref[...] = jnp.dot(S, Bm_ref[...], preferred_element_type=jnp.float32)
    R_ref[...] = jnp.dot(Xm, Wr_ref[...], preferred_element_type=jnp.float32)
    hh = hist_ref[...]
    c = hh - jnp.mean(hh, axis=-1, keepdims=True)
    denom = jnp.maximum(jnp.sqrt(jnp.mean(c * c, axis=-1, keepdims=True)),
                        1e-6)
    nrm_ref[...] = c / denom


_IMIN = -2147483648
_NG = 4
_GW = _N // _NG


def _topk_extract(scores, col_iota, diag, k):
    bits = lax.bitcast_convert_type(scores, jnp.int32)
    key = bits ^ (lax.shift_right_arithmetic(bits, 31) & 0x7FFFFFFF)
    comp = (key & -2048) | (2047 - col_iota)
    comp = jnp.where(diag, _IMIN, comp)
    ms = []
    for _ in range(k):
        mx = jnp.max(comp, axis=1, keepdims=True)
        ms.append(mx)
        comp = jnp.where(comp == mx, _IMIN, comp)
    mk = jnp.concatenate(ms, axis=1)
    idxs = 2047 - (mk & 2047)
    keyr = (mk & -2048) | 1024
    vbits = keyr ^ (lax.shift_right_arithmetic(keyr, 31) & 0x7FFFFFFF)
    vals = lax.bitcast_convert_type(vbits, jnp.float32)
    return vals, idxs


def _norm_chain(vsum):
    d1 = jnp.maximum(vsum, 1e-6)
    s1 = vsum / d1
    d2 = jnp.maximum(s1, 1e-6)
    scale = 1.0 / (d1 * d2)
    return scale, vsum * scale


def _main_body(mix_ref, SB_ref, R_ref, nrmb_ref, nrmf_ref,
               w_ref, idx_ref, *, gbase):
    ri = pl.program_id(0)
    row0 = ri * _BM
    col = lax.broadcasted_iota(jnp.int32, (_BM, _N), 1)
    rowg = lax.broadcasted_iota(jnp.int32, (_BM, 1), 0) + row0
    diag = col == rowg

    logits = lax.dot_general(SB_ref[0], R_ref[0],
                             (((1,), (1,)), ((), ())),
                             preferred_element_type=jnp.float32) / _TAU
    mrow = jnp.max(logits, axis=1, keepdims=True)
    z = jnp.sum(jnp.exp(logits - mrow), axis=1, keepdims=True)
    fv, fi = _topk_extract(logits, col, diag, _K)
    fv = jnp.exp(fv - mrow) / z
    fsum = jnp.sum(fv, axis=1, keepdims=True)
    fscale, fs2 = _norm_chain(fsum)

    sim = lax.dot_general(nrmb_ref[0], nrmf_ref[0],
                          (((1,), (1,)), ((), ())),
                          preferred_element_type=jnp.float32) * (1.0 / _L)
    rv, ri_ = _topk_extract(sim, col, diag, _K)
    rv = jnp.maximum(rv, 0.0)
    rsum = jnp.sum(rv, axis=1, keepdims=True)
    rscale, rs2 = _norm_chain(rsum)

    mix = jax.nn.sigmoid(mix_ref[0])
    fin = jnp.maximum(mix * fs2 + (1.0 - mix) * rs2, 1e-6)
    wf = fv * (fscale * mix / fin)
    wr = rv * (rscale * (1.0 - mix) / fin)

    w_ref[...] = jnp.concatenate([wf, wr], axis=1)
    idx_ref[...] = jnp.concatenate([fi, ri_], axis=1) + gbase


def _head_body(m_ref, W1_ref, b1_ref, W2_ref, b2_ref, y_ref):
    hh = jnp.maximum(jnp.dot(m_ref[...], W1_ref[...],
                             preferred_element_type=jnp.float32)
                     + b1_ref[...], 0.0)
    y = jnp.dot(hh, W2_ref[...], preferred_element_type=jnp.float32) + b2_ref[...]
    y_ref[...] = y[:, 0:1]


def _run_tc_stages(X, history, enc_W1, enc_b1, enc_W2, enc_b2, Ws, Wr, Bmat,
                   mix_logit, interpret=False):
    nb = _N // _BM
    prep = pl.pallas_call(
        _prep_body,
        grid=(_B, nb),
        in_specs=[
            pl.BlockSpec((1, _BM, _D), lambda b, i: (b, i, 0)),
            pl.BlockSpec((1, _BM, _L), lambda b, i: (b, i, 0)),
            pl.BlockSpec((_D, _BOT), lambda b, i: (0, 0)),
            pl.BlockSpec((_BOT,), lambda b, i: (0,)),
            pl.BlockSpec((_BOT, _D), lambda b, i: (0, 0)),
            pl.BlockSpec((_D,), lambda b, i: (0,)),
            pl.BlockSpec((_D, _RANK), lambda b, i: (0, 0)),
            pl.BlockSpec((_D, _RANK), lambda b, i: (0, 0)),
            pl.BlockSpec((_RANK, _RANK), lambda b, i: (0, 0)),
        ],
        out_specs=[
            pl.BlockSpec((1, _BM, _D), lambda b, i: (b, i, 0)),
            pl.BlockSpec((1, _BM, _RANK), lambda b, i: (b, i, 0)),
            pl.BlockSpec((1, _BM, _RANK), lambda b, i: (b, i, 0)),
            pl.BlockSpec((1, _BM, _L), lambda b, i: (b, i, 0)),
        ],
        out_shape=[
            jax.ShapeDtypeStruct((_B, _N, _D), jnp.float32),
            jax.ShapeDtypeStruct((_B, _N, _RANK), jnp.float32),
            jax.ShapeDtypeStruct((_B, _N, _RANK), jnp.float32),
            jax.ShapeDtypeStruct((_B, _N, _L), jnp.float32),
        ],
        interpret=interpret,
    )

    Xm, SB, R, nrm = prep(X, history, enc_W1, enc_b1, enc_W2, enc_b2,
                          Ws, Wr, Bmat)
    return Xm, SB, R, nrm


def _main_call_batch(b, mix1, SB, R, nrm, interpret=False):
    nb = _N // _BM
    body = functools.partial(_main_body, gbase=b * _N)
    main = pl.pallas_call(
        body,
        grid=(nb,),
        in_specs=[
            pl.BlockSpec(memory_space=pltpu.SMEM),
            pl.BlockSpec((1, _BM, _RANK), lambda i: (b, i, 0)),
            pl.BlockSpec((1, _N, _RANK), lambda i: (b, 0, 0)),
            pl.BlockSpec((1, _BM, _L), lambda i: (b, i, 0)),
            pl.BlockSpec((1, _N, _L), lambda i: (b, 0, 0)),
        ],
        out_specs=[
            pl.BlockSpec((_BM, 2 * _K), lambda i: (i, 0)),
            pl.BlockSpec((_BM, 2 * _K), lambda i: (i, 0)),
        ],
        out_shape=[
            jax.ShapeDtypeStruct((_N, 2 * _K), jnp.float32),
            jax.ShapeDtypeStruct((_N, 2 * _K), jnp.int32),
        ],
        interpret=interpret,
    )
    return main(mix1, SB, R, nrm, nrm)


def _head_call(m, h_W1, h_b1, h_W2, h_b2, interpret=False):
    rows = m.shape[0]
    nb = rows // 512
    head = pl.pallas_call(
        _head_body,
        grid=(nb,),
        in_specs=[
            pl.BlockSpec((512, _D), lambda i: (i, 0)),
            pl.BlockSpec((_D, _DH), lambda i: (0, 0)),
            pl.BlockSpec((_DH,), lambda i: (0,)),
            pl.BlockSpec((_DH, 1), lambda i: (0, 0)),
            pl.BlockSpec((1,), lambda i: (0,)),
        ],
        out_specs=pl.BlockSpec((512, 1), lambda i: (i, 0)),
        out_shape=jax.ShapeDtypeStruct((rows, 1), jnp.float32),
        interpret=interpret,
    )
    y = head(m, h_W1, h_b1, h_W2, h_b2)
    return y.reshape(rows)


_KT = 2 * _K
_NW = 32
_RPW = (_B * _N) // _NW
_SLAB = 64
_NSLAB = _RPW // _SLAB


def _sc_gather_body(Xm_hbm, idx_hbm, w_hbm, m_hbm,
                    idx_v, w_v, rows_a, rows_b, out_v, sem_a, sem_b,
                    *, rpw, nslab):
    nc = 2
    wid = lax.axis_index("s") * nc + lax.axis_index("c")
    base = wid * rpw

    def wait(buf, sem):
        pltpu.make_async_copy(Xm_hbm.at[pl.ds(0, 2 * _KT)], buf, sem).wait()

    def compute_pair(rm, pair):
        wb = rm * _KT
        wc = [w_v[pl.ds(wb + 16 * t, 16)] for t in range(5)]
        acc0 = [None] * 8
        acc1 = [None] * 8
        for j in range(_KT):
            wj0 = wc[j // 16][j % 16]
            wj1 = wc[(j + _KT) // 16][(j + _KT) % 16]
            for c in range(8):
                x0 = pair[j, pl.ds(c * 16, 16)]
                x1 = pair[_KT + j, pl.ds(c * 16, 16)]
                acc0[c] = wj0 * x0 if j == 0 else acc0[c] + wj0 * x0
                acc1[c] = wj1 * x1 if j == 0 else acc1[c] + wj1 * x1
        ob = rm * _D
        for c in range(8):
            out_v[pl.ds(ob + c * 16, 16)] = acc0[c]
            out_v[pl.ds(ob + _D + c * 16, 16)] = acc1[c]

    @pl.loop(0, nslab)
    def _slab(slab):
        sbase = base + slab * _SLAB
        pltpu.sync_copy(idx_hbm.at[pl.ds(sbase * _KT, _SLAB * _KT)], idx_v)
        pltpu.sync_copy(w_hbm.at[pl.ds(sbase * _KT, _SLAB * _KT)],
                        w_v.at[pl.ds(0, _SLAB * _KT)])
        pltpu.async_copy(Xm_hbm.at[idx_v.at[pl.ds(0, 2 * _KT)]],
                         rows_a, sem_a)
        pltpu.async_copy(Xm_hbm.at[idx_v.at[pl.ds(2 * _KT, 2 * _KT)]],
                         rows_b, sem_b)

        @pl.loop(0, _SLAB, step=4)
        def _rows(r):
            wait(rows_a, sem_a)
            compute_pair(r, rows_a)

            @pl.when(r + 4 < _SLAB)
            def _():
                pltpu.async_copy(
                    Xm_hbm.at[idx_v.at[pl.ds((r + 4) * _KT, 2 * _KT)]],
                    rows_a, sem_a)

            wait(rows_b, sem_b)
            compute_pair(r + 2, rows_b)

            @pl.when(r + 6 < _SLAB)
            def _():
                pltpu.async_copy(
                    Xm_hbm.at[idx_v.at[pl.ds((r + 6) * _KT, 2 * _KT)]],
                    rows_b, sem_b)

        pltpu.sync_copy(out_v, m_hbm.at[pl.ds(sbase * _D, _SLAB * _D)])


def _sc_gather(Xm2, idx40, w40, nrows):
    rpw = nrows // _NW
    body = functools.partial(_sc_gather_body, rpw=rpw,
                             nslab=max(1, rpw // _SLAB))
    mesh = plsc.VectorSubcoreMesh(core_axis_name="c", subcore_axis_name="s")
    f = pl.kernel(
        body,
        out_type=jax.ShapeDtypeStruct((nrows * _D,), jnp.float32),
        mesh=mesh,
        scratch_types=[
            pltpu.VMEM((_SLAB * _KT,), jnp.int32),
            pltpu.VMEM((_SLAB * _KT + 16,), jnp.float32),
            pltpu.VMEM((2 * _KT, _D), jnp.float32),
            pltpu.VMEM((2 * _KT, _D), jnp.float32),
            pltpu.VMEM((_SLAB * _D,), jnp.float32),
            pltpu.SemaphoreType.DMA,
            pltpu.SemaphoreType.DMA,
        ],
    )
    return f(Xm2, idx40.reshape(-1), w40.reshape(-1)).reshape(nrows, _D)


def kernel(X, history, enc_W1, enc_b1, enc_W2, enc_b2, Ws, Wr, Bmat,
           mix_logit, h_W1, h_b1, h_W2, h_b2):
    Xm, SB, R, nrm = _run_tc_stages(X, history, enc_W1, enc_b1, enc_W2,
                                    enc_b2, Ws, Wr, Bmat, mix_logit)
    Xm2 = Xm.reshape(_B * _N, _D)
    mix1 = mix_logit.reshape((1,))
    ys = []
    for b in range(_B):
        wb, ib = _main_call_batch(b, mix1, SB, R, nrm)
        m_b = _sc_gather(Xm2, ib, wb, _N)
        ys.append(_head_call(m_b, h_W1, h_b1, h_W2, h_b2))
    return jnp.stack(ys, axis=0)


# speedup vs baseline: 1.0538x; 1.0538x over previous
import functools
import math

import jax
import jax.numpy as jnp
from jax import lax
from jax.experimental import pallas as pl
from jax.experimental.pallas import tpu as pltpu
from jax.experimental.pallas import tpu_sc as plsc

_B, _N, _D, _L = 4, 2048, 128, 128
_RANK, _DH, _BOT = 32, 256, 64
_K = 20
_TAU = 1.0
_BM = 256


def _prep_body(X_ref, hist_ref, W1_ref, b1_ref, W2_ref, b2_ref, Ws_ref,
               Wr_ref, Bm_ref, Xm_ref, SB_ref, R_ref, nrm_ref):
    X = X_ref[...]
    h = jnp.maximum(jnp.dot(X, W1_ref[...], preferred_element_type=jnp.float32)
                    + b1_ref[...], 0.0)
    Xm = X + jnp.dot(h, W2_ref[...], preferred_element_type=jnp.float32) + b2_ref[...]
    Xm_ref[...] = Xm
    S = jnp.dot(Xm, Ws_ref[...], preferred_element_type=jnp.float32)
    SB_ref[...] = jnp.dot(S, Bm_ref[...], preferred_element_type=jnp.float32)
    R_ref[...] = jnp.dot(Xm, Wr_ref[...], preferred_element_type=jnp.float32)
    hh = hist_ref[...]
    c = hh - jnp.mean(hh, axis=-1, keepdims=True)
    denom = jnp.maximum(jnp.sqrt(jnp.mean(c * c, axis=-1, keepdims=True)),
                        1e-6)
    nrm_ref[...] = c / denom


_IMIN = -2147483648
def _topk_extract(scores, col_iota, diag, k):
    bits = lax.bitcast_convert_type(scores, jnp.int32)
    key = bits ^ (lax.shift_right_arithmetic(bits, 31) & 0x7FFFFFFF)
    comp = (key & -2048) | (2047 - col_iota)
    comp = jnp.where(diag, _IMIN, comp)
    ms = []
    for _ in range(k):
        mx = jnp.max(comp, axis=1, keepdims=True)
        ms.append(mx)
        comp = jnp.where(comp == mx, _IMIN, comp)
    mk = jnp.concatenate(ms, axis=1)
    idxs = 2047 - (mk & 2047)
    keyr = (mk & -2048) | 1024
    vbits = keyr ^ (lax.shift_right_arithmetic(keyr, 31) & 0x7FFFFFFF)
    vals = lax.bitcast_convert_type(vbits, jnp.float32)
    return vals, idxs


def _norm_chain(vsum):
    d1 = jnp.maximum(vsum, 1e-6)
    s1 = vsum / d1
    d2 = jnp.maximum(s1, 1e-6)
    scale = 1.0 / (d1 * d2)
    return scale, vsum * scale


def _main_body(mix_ref, SB_ref, R_ref, nrmb_ref, nrmf_ref,
               w_ref, idx_ref, *, gbase):
    ri = pl.program_id(0)
    row0 = ri * _BM
    col = lax.broadcasted_iota(jnp.int32, (_BM, _N), 1)
    rowg = lax.broadcasted_iota(jnp.int32, (_BM, 1), 0) + row0
    diag = col == rowg

    logits = lax.dot_general(SB_ref[0], R_ref[0],
                             (((1,), (1,)), ((), ())),
                             preferred_element_type=jnp.float32) / _TAU
    mrow = jnp.max(logits, axis=1, keepdims=True)
    z = jnp.sum(jnp.exp(logits - mrow), axis=1, keepdims=True)
    fv, fi = _topk_extract(logits, col, diag, _K)
    fv = jnp.exp(fv - mrow) / z
    fsum = jnp.sum(fv, axis=1, keepdims=True)
    fscale, fs2 = _norm_chain(fsum)

    sim = lax.dot_general(nrmb_ref[0], nrmf_ref[0],
                          (((1,), (1,)), ((), ())),
                          preferred_element_type=jnp.float32) * (1.0 / _L)
    rv, ri_ = _topk_extract(sim, col, diag, _K)
    rv = jnp.maximum(rv, 0.0)
    rsum = jnp.sum(rv, axis=1, keepdims=True)
    rscale, rs2 = _norm_chain(rsum)

    mix = jax.nn.sigmoid(mix_ref[0])
    fin = jnp.maximum(mix * fs2 + (1.0 - mix) * rs2, 1e-6)
    wf = fv * (fscale * mix / fin)
    wr = rv * (rscale * (1.0 - mix) / fin)

    w_ref[...] = jnp.concatenate([wf, wr], axis=1)
    idx_ref[...] = jnp.concatenate([fi, ri_], axis=1) + gbase


def _head_body(m_ref, W1_ref, b1_ref, W2_ref, b2_ref, y_ref):
    hh = jnp.maximum(jnp.dot(m_ref[...], W1_ref[...],
                             preferred_element_type=jnp.float32)
                     + b1_ref[...], 0.0)
    y = jnp.dot(hh, W2_ref[...], preferred_element_type=jnp.float32) + b2_ref[...]
    y_ref[...] = y[:, 0:1]


def _run_tc_stages(X, history, enc_W1, enc_b1, enc_W2, enc_b2, Ws, Wr, Bmat,
                   mix_logit, interpret=False):
    nb = _N // _BM
    prep = pl.pallas_call(
        _prep_body,
        grid=(_B, nb),
        in_specs=[
            pl.BlockSpec((1, _BM, _D), lambda b, i: (b, i, 0)),
            pl.BlockSpec((1, _BM, _L), lambda b, i: (b, i, 0)),
            pl.BlockSpec((_D, _BOT), lambda b, i: (0, 0)),
            pl.BlockSpec((_BOT,), lambda b, i: (0,)),
            pl.BlockSpec((_BOT, _D), lambda b, i: (0, 0)),
            pl.BlockSpec((_D,), lambda b, i: (0,)),
            pl.BlockSpec((_D, _RANK), lambda b, i: (0, 0)),
            pl.BlockSpec((_D, _RANK), lambda b, i: (0, 0)),
            pl.BlockSpec((_RANK, _RANK), lambda b, i: (0, 0)),
        ],
        out_specs=[
            pl.BlockSpec((1, _BM, _D), lambda b, i: (b, i, 0)),
            pl.BlockSpec((1, _BM, _RANK), lambda b, i: (b, i, 0)),
            pl.BlockSpec((1, _BM, _RANK), lambda b, i: (b, i, 0)),
            pl.BlockSpec((1, _BM, _L), lambda b, i: (b, i, 0)),
        ],
        out_shape=[
            jax.ShapeDtypeStruct((_B, _N, _D), jnp.float32),
            jax.ShapeDtypeStruct((_B, _N, _RANK), jnp.float32),
            jax.ShapeDtypeStruct((_B, _N, _RANK), jnp.float32),
            jax.ShapeDtypeStruct((_B, _N, _L), jnp.float32),
        ],
        interpret=interpret,
    )

    Xm, SB, R, nrm = prep(X, history, enc_W1, enc_b1, enc_W2, enc_b2,
                          Ws, Wr, Bmat)
    return Xm, SB, R, nrm


def _main_call_batch(b, mix1, SB, R, nrm, interpret=False):
    nb = _N // _BM
    body = functools.partial(_main_body, gbase=b * _N)
    main = pl.pallas_call(
        body,
        grid=(nb,),
        in_specs=[
            pl.BlockSpec(memory_space=pltpu.SMEM),
            pl.BlockSpec((1, _BM, _RANK), lambda i: (b, i, 0)),
            pl.BlockSpec((1, _N, _RANK), lambda i: (b, 0, 0)),
            pl.BlockSpec((1, _BM, _L), lambda i: (b, i, 0)),
            pl.BlockSpec((1, _N, _L), lambda i: (b, 0, 0)),
        ],
        out_specs=[
            pl.BlockSpec((_BM, 2 * _K), lambda i: (i, 0)),
            pl.BlockSpec((_BM, 2 * _K), lambda i: (i, 0)),
        ],
        out_shape=[
            jax.ShapeDtypeStruct((_N, 2 * _K), jnp.float32),
            jax.ShapeDtypeStruct((_N, 2 * _K), jnp.int32),
        ],
        interpret=interpret,
    )
    return main(mix1, SB, R, nrm, nrm)


def _head_call(m, h_W1, h_b1, h_W2, h_b2, interpret=False):
    rows = m.shape[0]
    nb = rows // 512
    head = pl.pallas_call(
        _head_body,
        grid=(nb,),
        in_specs=[
            pl.BlockSpec((512, _D), lambda i: (i, 0)),
            pl.BlockSpec((_D, _DH), lambda i: (0, 0)),
            pl.BlockSpec((_DH,), lambda i: (0,)),
            pl.BlockSpec((_DH, 1), lambda i: (0, 0)),
            pl.BlockSpec((1,), lambda i: (0,)),
        ],
        out_specs=pl.BlockSpec((512, 1), lambda i: (i, 0)),
        out_shape=jax.ShapeDtypeStruct((rows, 1), jnp.float32),
        interpret=interpret,
    )
    y = head(m, h_W1, h_b1, h_W2, h_b2)
    return y.reshape(rows)


_KT = 2 * _K
_NW = 32
_RPW = (_B * _N) // _NW
_SLAB = 64
_NSLAB = _RPW // _SLAB


def _sc_gather_body(Xm_hbm, idx_hbm, w_hbm, m_hbm,
                    idx_v, w_v, rows_a, rows_b, out_v, sem_a, sem_b,
                    *, rpw, nslab):
    nc = 2
    wid = lax.axis_index("s") * nc + lax.axis_index("c")
    base = wid * rpw

    def wait(buf, sem):
        pltpu.make_async_copy(Xm_hbm.at[pl.ds(0, _KT)], buf, sem).wait()

    def compute_row(rm, rows):
        wb = rm * _KT
        ob = rm * _D
        wc = [w_v[pl.ds(wb, 16)], w_v[pl.ds(wb + 16, 16)],
              w_v[pl.ds(wb + 32, 16)]]
        accs = [None] * 8
        for j in range(_KT):
            wj = wc[j // 16][j % 16]
            for c in range(8):
                x = rows[j, pl.ds(c * 16, 16)]
                accs[c] = wj * x if j == 0 else accs[c] + wj * x
        for c in range(8):
            out_v[pl.ds(ob + c * 16, 16)] = accs[c]

    @pl.loop(0, nslab)
    def _slab(slab):
        sbase = base + slab * _SLAB
        pltpu.sync_copy(idx_hbm.at[pl.ds(sbase, _SLAB)], idx_v)
        pltpu.sync_copy(w_hbm.at[pl.ds(sbase * _KT, _SLAB * _KT)],
                        w_v.at[pl.ds(0, _SLAB * _KT)])
        pltpu.async_copy(Xm_hbm.at[idx_v.at[0]], rows_a, sem_a)
        pltpu.async_copy(Xm_hbm.at[idx_v.at[1]], rows_b, sem_b)

        @pl.loop(0, _SLAB, step=2)
        def _rows(r):
            wait(rows_a, sem_a)
            compute_row(r, rows_a)

            @pl.when(r + 2 < _SLAB)
            def _():
                pltpu.async_copy(Xm_hbm.at[idx_v.at[r + 2]], rows_a, sem_a)

            wait(rows_b, sem_b)
            compute_row(r + 1, rows_b)

            @pl.when(r + 3 < _SLAB)
            def _():
                pltpu.async_copy(Xm_hbm.at[idx_v.at[r + 3]], rows_b, sem_b)

        pltpu.sync_copy(out_v, m_hbm.at[pl.ds(sbase * _D, _SLAB * _D)])


def _sc_gather(Xm2, idx40, w40, nrows):
    rpw = nrows // _NW
    body = functools.partial(_sc_gather_body, rpw=rpw,
                             nslab=max(1, rpw // _SLAB))
    mesh = plsc.VectorSubcoreMesh(core_axis_name="c", subcore_axis_name="s")
    f = pl.kernel(
        body,
        out_type=jax.ShapeDtypeStruct((nrows * _D,), jnp.float32),
        mesh=mesh,
        scratch_types=[
            pltpu.VMEM((_SLAB, _KT), jnp.int32),
            pltpu.VMEM((_SLAB * _KT + 16,), jnp.float32),
            pltpu.VMEM((_KT, _D), jnp.float32),
            pltpu.VMEM((_KT, _D), jnp.float32),
            pltpu.VMEM((_SLAB * _D,), jnp.float32),
            pltpu.SemaphoreType.DMA,
            pltpu.SemaphoreType.DMA,
        ],
    )
    return f(Xm2, idx40, w40.reshape(-1)).reshape(nrows, _D)


def kernel(X, history, enc_W1, enc_b1, enc_W2, enc_b2, Ws, Wr, Bmat,
           mix_logit, h_W1, h_b1, h_W2, h_b2):
    Xm, SB, R, nrm = _run_tc_stages(X, history, enc_W1, enc_b1, enc_W2,
                                    enc_b2, Ws, Wr, Bmat, mix_logit)
    Xm2 = Xm.reshape(_B * _N, _D)
    mix1 = mix_logit.reshape((1,))
    ys = []
    for b in range(_B):
        wb, ib = _main_call_batch(b, mix1, SB, R, nrm)
        m_b = _sc_gather(Xm2, ib, wb, _N)
        ys.append(_head_call(m_b, h_W1, h_b1, h_W2, h_b2))
    return jnp.stack(ys, axis=0)
